# baseline (speedup 1.0000x reference)
import jax
import jax.numpy as jnp
from jax import lax
from jax.experimental import pallas as pl
from jax.experimental.pallas import tpu as pltpu
from jax.experimental.pallas import tpu_sc as plsc

_D = 10
_ALPHA = 0.2
_ZPAD = 4000

_NC = 2
_NS = 16
_NW = _NC * _NS
_CH = 128
_GPC = 4
_CHUNK = _CH * _GPC

_G = 8
_GD = _G * _D
_NG = 50
_BR = _GD * _NG


def _matmul_body(nblocks, x_ref, w_ref, o_ref):
    val = lax.dot_general(
        x_ref[...], w_ref[...], (((1,), (0,)), ((), ())),
        preferred_element_type=jnp.float32)
    o_ref[...] = jnp.where(pl.program_id(0) < nblocks, val, 0.0)


def _matmul_padded(h, W):
    import functools
    n, f = h.shape
    block = 1000
    nblocks = n // block
    total = nblocks + _ZPAD // block
    return pl.pallas_call(
        functools.partial(_matmul_body, nblocks),
        grid=(total,),
        in_specs=[pl.BlockSpec((block, f),
                               lambda i, nb=nblocks - 1: (jnp.minimum(i, nb), 0)),
                  pl.BlockSpec((f, W.shape[1]), lambda i: (0, 0))],
        out_specs=pl.BlockSpec((block, W.shape[1]), lambda i: (i, 0)),
        out_shape=jax.ShapeDtypeStruct((n + _ZPAD, W.shape[1]), jnp.float32),
    )(h, W)


def _gather_body(table, gidx, out, idx_v, rows_v, sem):
    wid = lax.axis_index("s") * _NC + lax.axis_index("c")
    rows_per_worker = out.shape[0] // _NW
    nchunks = rows_per_worker // _CHUNK
    base0 = wid * rows_per_worker

    def body(c, carry):
        base = base0 + c * _CHUNK
        pltpu.sync_copy(gidx.at[pl.ds(base, _CHUNK)], idx_v)
        pltpu.async_copy(table.at[idx_v], rows_v, sem).wait()
        pltpu.sync_copy(rows_v, out.at[pl.ds(base, _CHUNK)])
        return carry

    lax.fori_loop(0, nchunks, body, 0)


def _sc_gather(table, gidx_flat, npad):
    f = table.shape[1]
    mesh = plsc.VectorSubcoreMesh(core_axis_name="c", subcore_axis_name="s")
    return pl.kernel(
        _gather_body,
        mesh=mesh,
        out_type=jax.ShapeDtypeStruct((npad, f), jnp.float32),
        scratch_types=[pltpu.VMEM((_CHUNK,), jnp.int32),
                       pltpu.VMEM((_CHUNK, f), jnp.float32),
                       pltpu.SemaphoreType.DMA],
    )(table, gidx_flat)


def _attn_body(x_ref, o_ref):
    r = lax.broadcasted_iota(jnp.int32, (_GD, _GD), 0) // _D
    c = lax.broadcasted_iota(jnp.int32, (_GD, _GD), 1) // _D
    m = r == c
    for g in range(_NG):
        xg = x_ref[pl.ds(g * _GD, _GD), :]
        s = lax.dot_general(xg, xg, (((1,), (1,)), ((), ())),
                            preferred_element_type=jnp.float32)
        l = jnp.where(s >= 0, s, _ALPHA * s)
        lm = jnp.where(m, l, -jnp.inf)
        cmax = jnp.max(lm, axis=0, keepdims=True)
        e = jnp.where(m, jnp.exp(l - cmax), 0.0)
        denom = jnp.sum(e, axis=0, keepdims=True)
        att = e / denom
        o = lax.dot_general(att, xg, (((1,), (0,)), ((), ())),
                            preferred_element_type=jnp.float32)
        fin = jnp.where(o > 0, o, jnp.exp(jnp.minimum(o, 0.0)) - 1.0)
        o_ref[pl.ds(g * _G, _G), :, :] = fin.reshape(_G, _D, fin.shape[1])


def _attention(hwd, n):
    f = hwd.shape[1]
    grid = (n * _D) // _BR
    bn = _BR // _D
    return pl.pallas_call(
        _attn_body,
        grid=(grid,),
        in_specs=[pl.BlockSpec((_BR, f), lambda i: (i, 0))],
        out_specs=pl.BlockSpec((bn, _D, f), lambda i: (i, 0, 0)),
        out_shape=jax.ShapeDtypeStruct((n, _D, f), jnp.float32),
    )(hwd)


def kernel(h, edge_feats, edge_indices, adj, W, W_edge):
    n, f = h.shape
    e = edge_indices.shape[1]

    src = edge_indices[0]
    dst = edge_indices[1]
    src_s, dst_s = lax.sort((src, dst), num_keys=1, is_stable=True)
    iota = jnp.arange(e, dtype=jnp.int32)
    is_first = jnp.concatenate(
        [jnp.ones((1,), bool), src_s[1:] != src_s[:-1]])
    first = lax.cummax(jnp.where(is_first, iota, 0), axis=0)
    pos = iota - first
    is_last = jnp.concatenate(
        [src_s[1:] != src_s[:-1], jnp.ones((1,), bool)])
    valid = (pos < _D - 1) | is_last
    posc = jnp.minimum(pos, _D - 1)
    rowi = jnp.where(valid, src_s, n)
    sent = n + (jnp.arange((n + 1) * _D, dtype=jnp.int32) % _ZPAD).reshape(
        n + 1, _D)
    vali = jnp.where(valid, dst_s.astype(jnp.int32),
                     n + (jnp.arange(e, dtype=jnp.int32) % _ZPAD))
    gidx = sent.at[rowi, posc].set(vali)
    gidx_flat = gidx[:n].reshape(n * _D)
    shard = _NW * _CHUNK
    npad = ((n * _D + shard - 1) // shard) * shard
    gidx_flat = jnp.concatenate(
        [gidx_flat,
         n + (jnp.arange(npad - n * _D, dtype=jnp.int32) % _ZPAD)])

    table = _matmul_padded(h, W)
    hwd = _sc_gather(table, gidx_flat, npad)
    return _attention(hwd, n)


# speedup vs baseline: 1.6920x; 1.6920x over previous
import jax
import jax.numpy as jnp
from jax import lax
from jax.experimental import pallas as pl
from jax.experimental.pallas import tpu as pltpu
from jax.experimental.pallas import tpu_sc as plsc

_D = 10
_ALPHA = 0.2
_ZPAD = 4000

_NC = 2
_NS = 16
_NW = _NC * _NS
_CH = 128
_GPC = 4
_CHUNK = _CH * _GPC

_G = 16
_GD = _G * _D
_NG = 25
_BR = _GD * _NG


def _matmul_body(nblocks, x_ref, w_ref, o_ref):
    val = lax.dot_general(
        x_ref[...], w_ref[...], (((1,), (0,)), ((), ())),
        preferred_element_type=jnp.float32)
    o_ref[...] = jnp.where(pl.program_id(0) < nblocks, val, 0.0)


def _matmul_padded(h, W):
    import functools
    n, f = h.shape
    block = 1000
    nblocks = n // block
    total = nblocks + _ZPAD // block
    return pl.pallas_call(
        functools.partial(_matmul_body, nblocks),
        grid=(total,),
        in_specs=[pl.BlockSpec((block, f),
                               lambda i, nb=nblocks - 1: (jnp.minimum(i, nb), 0)),
                  pl.BlockSpec((f, W.shape[1]), lambda i: (0, 0))],
        out_specs=pl.BlockSpec((block, W.shape[1]), lambda i: (i, 0)),
        out_shape=jax.ShapeDtypeStruct((n + _ZPAD, W.shape[1]), jnp.float32),
    )(h, W)


def _gather_body(table, gidx, out, idx_v, rows_v, sem):
    wid = lax.axis_index("s") * _NC + lax.axis_index("c")
    rows_per_worker = out.shape[0] // _NW
    nchunks = rows_per_worker // _CHUNK
    base0 = wid * rows_per_worker

    def body(c, carry):
        base = base0 + c * _CHUNK
        pltpu.sync_copy(gidx.at[pl.ds(base, _CHUNK)], idx_v)
        pltpu.async_copy(table.at[idx_v], rows_v, sem).wait()
        pltpu.sync_copy(rows_v, out.at[pl.ds(base, _CHUNK)])
        return carry

    lax.fori_loop(0, nchunks, body, 0)


def _sc_gather(table, gidx_flat, npad):
    f = table.shape[1]
    mesh = plsc.VectorSubcoreMesh(core_axis_name="c", subcore_axis_name="s")
    return pl.kernel(
        _gather_body,
        mesh=mesh,
        out_type=jax.ShapeDtypeStruct((npad, f), jnp.float32),
        scratch_types=[pltpu.VMEM((_CHUNK,), jnp.int32),
                       pltpu.VMEM((_CHUNK, f), jnp.float32),
                       pltpu.SemaphoreType.DMA],
    )(table, gidx_flat)


def _attn_body(x_ref, o_ref):
    r = lax.broadcasted_iota(jnp.int32, (_GD, _GD), 0) // _D
    c = lax.broadcasted_iota(jnp.int32, (_GD, _GD), 1) // _D
    m = r == c
    for g in range(_NG):
        xg = x_ref[pl.ds(g * _GD, _GD), :]
        s = lax.dot_general(xg, xg, (((1,), (1,)), ((), ())),
                            preferred_element_type=jnp.float32)
        l = jnp.where(s >= 0, s, _ALPHA * s)
        lm = jnp.where(m, l, -jnp.inf)
        cmax = jnp.max(lm, axis=0, keepdims=True)
        e = jnp.where(m, jnp.exp(l - cmax), 0.0)
        denom = jnp.sum(e, axis=0, keepdims=True)
        att = e / denom
        o = lax.dot_general(att, xg, (((1,), (0,)), ((), ())),
                            preferred_element_type=jnp.float32)
        fin = jnp.where(o > 0, o, jnp.exp(jnp.minimum(o, 0.0)) - 1.0)
        o_ref[pl.ds(g * _G, _G), :, :] = fin.reshape(_G, _D, fin.shape[1])


def _attention(hwd, n):
    f = hwd.shape[1]
    grid = (n * _D) // _BR
    bn = _BR // _D
    return pl.pallas_call(
        _attn_body,
        grid=(grid,),
        in_specs=[pl.BlockSpec((_BR, f), lambda i: (i, 0))],
        out_specs=pl.BlockSpec((bn, _D, f), lambda i: (i, 0, 0)),
        out_shape=jax.ShapeDtypeStruct((n, _D, f), jnp.float32),
    )(hwd)


def kernel(h, edge_feats, edge_indices, adj, W, W_edge):
    n, f = h.shape
    e = edge_indices.shape[1]

    src = edge_indices[0]
    dst = edge_indices[1]
    src_s, dst_s = lax.sort((src, dst), num_keys=1, is_stable=True)
    iota = jnp.arange(e, dtype=jnp.int32)
    is_first = jnp.concatenate(
        [jnp.ones((1,), bool), src_s[1:] != src_s[:-1]])
    first = lax.cummax(jnp.where(is_first, iota, 0), axis=0)
    pos = iota - first
    is_last = jnp.concatenate(
        [src_s[1:] != src_s[:-1], jnp.ones((1,), bool)])
    valid = (pos < _D - 1) | is_last
    posc = jnp.minimum(pos, _D - 1)
    shard = _NW * _CHUNK
    npad = ((n * _D + shard - 1) // shard) * shard
    total = max(npad, n * _D + e)
    sent = n + (jnp.arange(total, dtype=jnp.int32) % _ZPAD)
    tgt = jnp.where(valid, src_s * _D + posc, n * _D + iota)
    vali = jnp.where(valid, dst_s.astype(jnp.int32), n)
    flat = sent.at[tgt].set(vali, unique_indices=True)
    gidx_flat = jnp.concatenate(
        [flat[:n * _D],
         n + (jnp.arange(npad - n * _D, dtype=jnp.int32) % _ZPAD)])

    table = _matmul_padded(h, W)
    hwd = _sc_gather(table, gidx_flat, npad)
    return _attention(hwd, n)
